# baseline (speedup 1.0000x reference)
import functools

import jax
import jax.numpy as jnp
from jax import lax
from jax.experimental import pallas as pl
from jax.experimental.pallas import tpu as pltpu
from jax.experimental.pallas import tpu_sc as plsc

N = 65536
K = 1024
D = 32
COMMITMENT_COST = 0.25

B = 8192
G = N // B


def _vq_tc_body(x_ref, wt_ref, idx_ref, loss_ref, perp_ref, cnt_ref):
    step = pl.program_id(0)

    x = x_ref[...]
    wt = wt_ref[...]

    x2 = jnp.sum(x * x, axis=1, keepdims=True)
    e2 = jnp.sum(wt * wt, axis=0, keepdims=True)
    m2 = lax.dot_general(x, wt + wt, (((1,), (0,)), ((), ())),
                         preferred_element_type=jnp.float32)
    d = (x2 + e2) - m2

    dmin = jnp.min(d, axis=1, keepdims=True)
    iotaf = lax.broadcasted_iota(jnp.int32, (1, K), 1).astype(jnp.float32)
    maskedf = jnp.where(d == dmin, iotaf, float(K))
    idxf = jnp.min(maskedf, axis=1)
    idx_ref[0, 0, :] = idxf.astype(jnp.int32)

    onehot = (iotaf == idxf[:, None]).astype(jnp.float32)
    cnt = lax.dot_general(jnp.ones((1, B), jnp.float32), onehot,
                          (((1,), (0,)), ((), ())),
                          preferred_element_type=jnp.float32)

    @pl.when(step == 0)
    def _init():
        loss_ref[...] = jnp.zeros_like(loss_ref)
        cnt_ref[...] = jnp.zeros_like(cnt_ref)

    loss_ref[...] += jnp.sum(dmin)[None, None]
    cnt_ref[...] += cnt

    @pl.when(step == G - 1)
    def _finalize():
        loss_ref[...] = loss_ref[...] * ((1.0 + COMMITMENT_COST) / (N * D))
        avg = cnt_ref[...] * (1.0 / N)
        ent = -jnp.sum(avg * jnp.log(avg + 1e-10))
        perp_ref[...] = (jnp.exp(ent) * (1.0 / K))[None, None]


def _vq_stats(inputs, wt, interpret=False):
    return pl.pallas_call(
        _vq_tc_body,
        grid=(G,),
        in_specs=[
            pl.BlockSpec((B, D), lambda i: (i, 0)),
            pl.BlockSpec((D, K), lambda i: (0, 0)),
        ],
        out_specs=[
            pl.BlockSpec((1, 1, B), lambda i: (i, 0, 0)),
            pl.BlockSpec((1, 1), lambda i: (0, 0)),
            pl.BlockSpec((1, 1), lambda i: (0, 0)),
        ],
        out_shape=[
            jax.ShapeDtypeStruct((G, 1, B), jnp.int32),
            jax.ShapeDtypeStruct((1, 1), jnp.float32),
            jax.ShapeDtypeStruct((1, 1), jnp.float32),
        ],
        scratch_shapes=[pltpu.VMEM((1, K), jnp.float32)],
        compiler_params=pltpu.CompilerParams(
            dimension_semantics=("arbitrary",)),
        interpret=interpret,
    )(inputs, wt)


def _make_gather():
    info = plsc.get_sparse_core_info()
    nc, ns = info.num_cores, info.num_subcores
    nw = nc * ns
    bpw = N // nw
    mesh = plsc.VectorSubcoreMesh(core_axis_name="c", subcore_axis_name="s")

    @functools.partial(
        pl.kernel, mesh=mesh,
        out_type=jax.ShapeDtypeStruct((N, D), jnp.float32),
        scratch_types=[
            pltpu.VMEM((bpw,), jnp.int32),
            pltpu.VMEM((bpw // 2, D), jnp.float32),
            pltpu.VMEM((bpw // 2, D), jnp.float32),
            pltpu.SemaphoreType.DMA,
            pltpu.SemaphoreType.DMA,
        ],
        compiler_params=pltpu.CompilerParams(use_tc_tiling_on_sc=False),
    )
    def gather_k(table_hbm, idx_hbm, out_hbm, idx_v, rows0, rows1, sg, sw):
        wid = lax.axis_index("s") * nc + lax.axis_index("c")
        base = wid * bpw
        h = bpw // 2
        pltpu.sync_copy(idx_hbm.at[pl.ds(base, bpw)], idx_v)
        g0 = pltpu.async_copy(table_hbm.at[idx_v.at[pl.ds(0, h)]], rows0, sg)
        g1 = pltpu.async_copy(table_hbm.at[idx_v.at[pl.ds(h, h)]], rows1, sg)
        g0.wait()
        w0 = pltpu.async_copy(rows0, out_hbm.at[pl.ds(base, h)], sw)
        g1.wait()
        w1 = pltpu.async_copy(rows1, out_hbm.at[pl.ds(base + h, h)], sw)
        w0.wait()
        w1.wait()

    return gather_k


def kernel(inputs, weight):
    wt = weight.T
    idx3, loss, perp = _vq_stats(inputs, wt)
    idx_flat = idx3.reshape(N)
    quantized = _make_gather()(weight, idx_flat)
    return (loss[0, 0], quantized, perp[0, 0], idx_flat[:, None])


# speedup vs baseline: 1.0088x; 1.0088x over previous
import functools

import jax
import jax.numpy as jnp
from jax import lax
from jax.experimental import pallas as pl
from jax.experimental.pallas import tpu as pltpu
from jax.experimental.pallas import tpu_sc as plsc

N = 65536
K = 1024
D = 32
COMMITMENT_COST = 0.25

B = 8192
G = N // B


def _vq_tc_body(x_ref, wt_ref, idx_ref, loss_ref, perp_ref, cnt_ref):
    step = pl.program_id(0)

    x = x_ref[...]
    wt = wt_ref[...]

    x2 = jnp.sum(x * x, axis=1, keepdims=True)
    e2 = jnp.sum(wt * wt, axis=0, keepdims=True)
    m2 = lax.dot_general(x, wt + wt, (((1,), (0,)), ((), ())),
                         preferred_element_type=jnp.float32)
    d = (x2 + e2) - m2

    dmin = jnp.min(d, axis=1, keepdims=True)
    iotaf = lax.broadcasted_iota(jnp.int32, (1, K), 1).astype(jnp.float32)
    maskedf = jnp.where(d == dmin, iotaf, float(K))
    idxf = jnp.min(maskedf, axis=1)
    idx_ref[0, 0, :] = idxf.astype(jnp.int32)

    onehot = (iotaf == idxf[:, None]).astype(jnp.float32)
    cnt = lax.dot_general(jnp.ones((1, B), jnp.float32), onehot,
                          (((1,), (0,)), ((), ())),
                          preferred_element_type=jnp.float32)

    @pl.when(step == 0)
    def _init():
        loss_ref[...] = jnp.zeros_like(loss_ref)
        cnt_ref[...] = jnp.zeros_like(cnt_ref)

    loss_ref[...] += jnp.sum(dmin)[None, None]
    cnt_ref[...] += cnt

    @pl.when(step == G - 1)
    def _finalize():
        loss_ref[...] = loss_ref[...] * ((1.0 + COMMITMENT_COST) / (N * D))
        avg = cnt_ref[...] * (1.0 / N)
        ent = -jnp.sum(avg * jnp.log(avg + 1e-10))
        perp_ref[...] = (jnp.exp(ent) * (1.0 / K))[None, None]


def _vq_stats(inputs, wt, interpret=False):
    return pl.pallas_call(
        _vq_tc_body,
        grid=(G,),
        in_specs=[
            pl.BlockSpec((B, D), lambda i: (i, 0)),
            pl.BlockSpec((D, K), lambda i: (0, 0)),
        ],
        out_specs=[
            pl.BlockSpec((1, 1, B), lambda i: (i, 0, 0)),
            pl.BlockSpec((1, 1), lambda i: (0, 0)),
            pl.BlockSpec((1, 1), lambda i: (0, 0)),
        ],
        out_shape=[
            jax.ShapeDtypeStruct((G, 1, B), jnp.int32),
            jax.ShapeDtypeStruct((1, 1), jnp.float32),
            jax.ShapeDtypeStruct((1, 1), jnp.float32),
        ],
        scratch_shapes=[pltpu.VMEM((1, K), jnp.float32)],
        compiler_params=pltpu.CompilerParams(
            dimension_semantics=("arbitrary",)),
        interpret=interpret,
    )(inputs, wt)


def _make_gather():
    info = plsc.get_sparse_core_info()
    nc, ns = info.num_cores, info.num_subcores
    nw = nc * ns
    bpw = N // nw
    mesh = plsc.VectorSubcoreMesh(core_axis_name="c", subcore_axis_name="s")

    @functools.partial(
        pl.kernel, mesh=mesh,
        out_type=jax.ShapeDtypeStruct((N, D), jnp.float32),
        scratch_types=[
            pltpu.VMEM((bpw,), jnp.int32),
            pltpu.VMEM((bpw, D), jnp.float32),
            pltpu.SemaphoreType.DMA,
        ],
        compiler_params=pltpu.CompilerParams(use_tc_tiling_on_sc=False),
    )
    def gather_k(table_hbm, idx_hbm, out_hbm, idx_v, rows_v, sem):
        wid = lax.axis_index("s") * nc + lax.axis_index("c")
        base = wid * bpw
        pltpu.sync_copy(idx_hbm.at[pl.ds(base, bpw)], idx_v)
        pltpu.async_copy(table_hbm.at[idx_v], rows_v, sem).wait()
        pltpu.sync_copy(rows_v, out_hbm.at[pl.ds(base, bpw)])

    return gather_k


def kernel(inputs, weight):
    wt = weight.T
    idx3, loss, perp = _vq_stats(inputs, wt)
    idx_flat = idx3.reshape(N)
    quantized = _make_gather()(weight, idx_flat)
    return (loss[0, 0], quantized, perp[0, 0], idx_flat[:, None])
